# baseline (speedup 1.0000x reference)
import functools

import jax
import jax.numpy as jnp
import numpy as np
from jax.experimental import pallas as pl
from jax.experimental.pallas import tpu as pltpu

_VMEM_LIMIT = 48 * 1024 * 1024


def _rup(v, m):
    return (v + m - 1) // m * m


def _band_stage_kernel(x_ref, bd1_ref, sh1_ref, bd2_ref, sh2_ref, e_ref,
                       q_ref, d_ref, p_ref, *, cin, cout, th, wid, hpa, nt):
    i = pl.program_id(1)
    xc = jnp.concatenate(
        [x_ref[0, pl.ds(ci * hpa + i * th, th + 4), :] for ci in range(cin)],
        axis=0)
    acc = None
    for dx in range(3):
        part = jnp.dot(bd1_ref[dx], xc[:, dx:dx + wid],
                       preferred_element_type=jnp.float32)
        acc = part if acc is None else acc + part
    y1 = jnp.maximum(acc + sh1_ref[0], 0.0).astype(jnp.bfloat16)
    zc = jnp.zeros((cout * (th + 2), 1), jnp.bfloat16)
    y1p = jnp.concatenate([zc, y1, zc], axis=1)
    acc2 = None
    for dx in range(3):
        part = jnp.dot(bd2_ref[dx], y1p[:, dx:dx + wid],
                       preferred_element_type=jnp.float32)
        acc2 = part if acc2 is None else acc2 + part
    y2 = jnp.maximum(acc2 + sh2_ref[...], 0.0)
    d_ref[0] = y2.reshape(cout, th, wid)
    y2b = y2.astype(jnp.bfloat16)
    m = jnp.maximum(
        jnp.dot(q_ref[0], y2b, preferred_element_type=jnp.float32),
        jnp.dot(q_ref[1], y2b, preferred_element_type=jnp.float32)
    ).astype(jnp.bfloat16)
    pooled = jnp.maximum(
        jnp.dot(m, e_ref[0], preferred_element_type=jnp.float32),
        jnp.dot(m, e_ref[1], preferred_element_type=jnp.float32))
    p_ref[0] = pooled.reshape(cout, th // 2, wid // 2).astype(jnp.bfloat16)


def _band_weights(w, scale, ro, ri, cin, cout, interleaved_cols):
    ws = w * scale[None, None, None, :]
    bds = []
    for dx in range(3):
        if interleaved_cols:
            acc = jnp.zeros((cout, ro, ri, cin), jnp.float32)
        else:
            acc = jnp.zeros((cout, ro, cin, ri), jnp.float32)
        for dy in range(3):
            e = jnp.eye(ro, ri, k=dy, dtype=jnp.float32)
            if interleaved_cols:
                acc = acc + (e[None, :, :, None]
                             * ws[dy, dx].T[:, None, None, :])
            else:
                acc = acc + (e[None, :, None, :]
                             * ws[dy, dx].T[:, None, :, None])
        bds.append(acc.reshape(cout * ro, cin * ri))
    return jnp.stack(bds).astype(jnp.bfloat16)


def _down_stage_band(x, w1, s1, b1, w2, s2, b2, *, cin, cout, h, th):
    n = x.shape[0]
    wid = h
    hpa = _rup(h + 4, 16)
    xp = jnp.pad(x.astype(jnp.bfloat16),
                 ((0, 0), (0, 0), (2, hpa - h - 2), (1, 1)))
    xp = xp.reshape(n, cin * hpa, wid + 2)
    bd1 = _band_weights(w1, s1, th + 2, th + 4, cin, cout, False)
    bd2 = _band_weights(w2, s2, th, th + 2, cout, cout, False)
    nt = h // th
    ro = (th + 2) * cout
    base = jnp.repeat(b1, th + 2).reshape(ro, 1)
    rl = np.arange(ro) % (th + 2)
    sh_first = jnp.where(jnp.asarray(rl == 0)[:, None], -1e30, base)
    sh_last = jnp.where(jnp.asarray(rl == th + 1)[:, None], -1e30, base)
    sh1 = jnp.concatenate([sh_first[None],
                           jnp.broadcast_to(base, (nt - 2, ro, 1)),
                           sh_last[None]], axis=0)
    sh2 = jnp.repeat(b2, th).reshape(-1, 1)
    we = np.zeros((2, wid, wid // 2), np.float32)
    we[0, np.arange(0, wid, 2), np.arange(wid // 2)] = 1
    we[1, np.arange(1, wid, 2), np.arange(wid // 2)] = 1
    e_sel = jnp.asarray(we, jnp.bfloat16)
    rq = np.zeros((2, cout * th // 2, cout * th), np.float32)
    rows = np.arange(cout * th // 2)
    src = (rows // (th // 2)) * th + 2 * (rows % (th // 2))
    rq[0, rows, src] = 1
    rq[1, rows, src + 1] = 1
    q_sel = jnp.asarray(rq, jnp.bfloat16)
    kfn = functools.partial(_band_stage_kernel, cin=cin, cout=cout, th=th,
                            wid=wid, hpa=hpa, nt=nt)
    d, p = pl.pallas_call(
        kfn,
        out_shape=(jax.ShapeDtypeStruct((n, cout, h, wid), jnp.float32),
                   jax.ShapeDtypeStruct((n, cout, h // 2, wid // 2),
                                        jnp.bfloat16)),
        grid=(n, nt),
        in_specs=[
            pl.BlockSpec((1, cin * hpa, wid + 2), lambda b, i: (b, 0, 0)),
            pl.BlockSpec((3, (th + 2) * cout, (th + 4) * cin),
                         lambda b, i: (0, 0, 0)),
            pl.BlockSpec((1, (th + 2) * cout, 1), lambda b, i: (i, 0, 0)),
            pl.BlockSpec((3, th * cout, (th + 2) * cout),
                         lambda b, i: (0, 0, 0)),
            pl.BlockSpec((th * cout, 1), lambda b, i: (0, 0)),
            pl.BlockSpec((2, wid, wid // 2), lambda b, i: (0, 0, 0)),
            pl.BlockSpec((2, cout * th // 2, cout * th),
                         lambda b, i: (0, 0, 0)),
        ],
        out_specs=(
            pl.BlockSpec((1, cout, th, wid), lambda b, i: (b, 0, i, 0)),
            pl.BlockSpec((1, cout, th // 2, wid // 2),
                         lambda b, i: (b, 0, i, 0)),
        ),
        compiler_params=pltpu.CompilerParams(
            dimension_semantics=("parallel", "arbitrary"),
            vmem_limit_bytes=_VMEM_LIMIT),
    )(xp, bd1, sh1, bd2, sh2, e_sel, q_sel)
    return d, p


def _flat_conv_t(xt, w_ref, sh_ref, col, *, h, w, cin, cout):
    m = h * w
    zp = jnp.zeros((cin, w + 1), jnp.bfloat16)
    xpf = jnp.concatenate([zp, xt, zp], axis=1)
    acc = None
    for dy in range(3):
        for dx in range(3):
            o = dy * w + dx
            sl = jax.lax.slice(xpf, (0, o), (cin, o + m))
            if dx == 0:
                sl = jnp.where(col == 0, jnp.bfloat16(0), sl)
            elif dx == 2:
                sl = jnp.where(col == w - 1, jnp.bfloat16(0), sl)
            part = jnp.dot(w_ref[3 * dy + dx], sl,
                           preferred_element_type=jnp.float32)
            acc = part if acc is None else acc + part
    return jnp.maximum(acc + sh_ref[...], 0.0)


def _deep_stage_kernel(x_ref, w1_ref, sh1_ref, w2_ref, sh2_ref, *rest,
                       h, w, cin, cout, pool):
    xt = x_ref[0]
    m = h * w
    col = jax.lax.broadcasted_iota(jnp.int32, (1, m), 1) % w
    y1 = _flat_conv_t(xt, w1_ref, sh1_ref, col,
                      h=h, w=w, cin=cin, cout=cout).astype(jnp.bfloat16)
    y2 = _flat_conv_t(y1, w2_ref, sh2_ref, col,
                      h=h, w=w, cin=cout, cout=cout)
    if pool:
        sel_ref, d_ref, p_ref = rest
        d_ref[0] = y2
        zc = jnp.zeros((cout, 1), jnp.float32)
        right = jnp.concatenate([jax.lax.slice(y2, (0, 1), (cout, m)), zc],
                                axis=1)
        mw = jnp.maximum(y2, right)
        zw = jnp.zeros((cout, w), jnp.float32)
        below = jnp.concatenate([jax.lax.slice(mw, (0, w), (cout, m)), zw],
                                axis=1)
        mh = jnp.maximum(mw, below)
        p_ref[0] = jnp.dot(mh.astype(jnp.bfloat16), sel_ref[...],
                           preferred_element_type=jnp.float32
                           ).astype(jnp.bfloat16)
    else:
        d_ref = rest[0]
        d_ref[0] = y2


def _pack_w_t(w, scale):
    ws = w * scale[None, None, None, :]
    return ws.reshape(9, w.shape[2], w.shape[3]).transpose(0, 2, 1).astype(
        jnp.bfloat16)


def _deep_stage(xt, w1, s1, b1, w2, s2, b2, *, h, w, pool):
    n, cin, m = xt.shape
    cout = w1.shape[-1]
    p1, p2 = _pack_w_t(w1, s1), _pack_w_t(w2, s2)
    sh1 = b1.reshape(cout, 1).astype(jnp.float32)
    sh2 = b2.reshape(cout, 1).astype(jnp.float32)
    out_shape = [jax.ShapeDtypeStruct((n, cout, m), jnp.float32)]
    out_specs = [pl.BlockSpec((1, cout, m), lambda b: (b, 0, 0))]
    in_specs = [
        pl.BlockSpec((1, cin, m), lambda b: (b, 0, 0)),
        pl.BlockSpec((9, cout, cin), lambda b: (0, 0, 0)),
        pl.BlockSpec((cout, 1), lambda b: (0, 0)),
        pl.BlockSpec((9, cout, cout), lambda b: (0, 0, 0)),
        pl.BlockSpec((cout, 1), lambda b: (0, 0)),
    ]
    args = [xt, p1, sh1, p2, sh2]
    if pool:
        out_shape.append(jax.ShapeDtypeStruct((n, cout, m // 4),
                                              jnp.bfloat16))
        out_specs.append(pl.BlockSpec((1, cout, m // 4), lambda b: (b, 0, 0)))
        in_specs.append(pl.BlockSpec((m, m // 4), lambda b: (0, 0)))
        q = np.arange(m // 4)
        p_src = 2 * (q // (w // 2)) * w + 2 * (q % (w // 2))
        selm = np.zeros((m, m // 4), np.float32)
        selm[p_src, q] = 1.0
        args.append(jnp.asarray(selm, jnp.bfloat16))
    kfn = functools.partial(_deep_stage_kernel, h=h, w=w, cin=cin, cout=cout,
                            pool=pool)
    outs = pl.pallas_call(
        kfn,
        out_shape=tuple(out_shape),
        grid=(n,),
        in_specs=in_specs,
        out_specs=tuple(out_specs),
        compiler_params=pltpu.CompilerParams(
            dimension_semantics=("parallel",),
            vmem_limit_bytes=_VMEM_LIMIT),
    )(*args)
    return (outs[0], outs[1]) if pool else (outs[0], None)


def _leaky(v):
    return jnp.where(v > 0, v, 0.2 * v)


def _head(x, o1, o2, o3):
    w, s, b = o1
    y = jax.lax.conv_general_dilated(
        x, w, (2, 2), ((1, 1), (1, 1)),
        dimension_numbers=("NHWC", "HWIO", "NHWC"))
    y = _leaky(y * s + b)
    w, s, b = o2
    y = jax.lax.conv_general_dilated(
        y, w, (1, 1), "VALID",
        dimension_numbers=("NHWC", "HWIO", "NHWC"))
    y = _leaky(y * s + b)
    w, s, b = o3
    y = jax.lax.conv_general_dilated(
        y, w, (1, 1), "VALID",
        dimension_numbers=("NHWC", "HWIO", "NHWC"))
    y = jax.nn.sigmoid(y * s + b)
    return jnp.transpose(y, (0, 3, 1, 2))


def kernel(img,
           d0_1_w, d0_1_scale, d0_1_shift, d0_2_w, d0_2_scale, d0_2_shift,
           d1_1_w, d1_1_scale, d1_1_shift, d1_2_w, d1_2_scale, d1_2_shift,
           d2_1_w, d2_1_scale, d2_1_shift, d2_2_w, d2_2_scale, d2_2_shift,
           d3_1_w, d3_1_scale, d3_1_shift, d3_2_w, d3_2_scale, d3_2_shift,
           d4_1_w, d4_1_scale, d4_1_shift, d4_2_w, d4_2_scale, d4_2_shift,
           d5_1_w, d5_1_scale, d5_1_shift, d5_2_w, d5_2_scale, d5_2_shift,
           c1_w, c1_scale, c1_shift, c2_w, c2_scale, c2_shift,
           o1_w, o1_scale, o1_shift, o2_w, o2_scale, o2_shift,
           o3_w, o3_scale, o3_shift):
    n = img.shape[0]
    downs = []

    band = [(3, 4, 512, 64, d0_1_w, d0_1_scale, d0_1_shift,
             d0_2_w, d0_2_scale, d0_2_shift),
            (4, 8, 256, 64, d1_1_w, d1_1_scale, d1_1_shift,
             d1_2_w, d1_2_scale, d1_2_shift),
            (8, 16, 128, 32, d2_1_w, d2_1_scale, d2_1_shift,
             d2_2_w, d2_2_scale, d2_2_shift)]
    x = img
    for cin, cout, h, th, w1, s1, b1, w2, s2, b2 in band:
        d, x = _down_stage_band(x, w1, s1, b1, w2, s2, b2,
                                cin=cin, cout=cout, h=h, th=th)
        downs.append(d)

    xt = x.reshape(n, 16, 64 * 64)
    deep = [(64, d3_1_w, d3_1_scale, d3_1_shift, d3_2_w, d3_2_scale, d3_2_shift),
            (32, d4_1_w, d4_1_scale, d4_1_shift, d4_2_w, d4_2_scale, d4_2_shift),
            (16, d5_1_w, d5_1_scale, d5_1_shift, d5_2_w, d5_2_scale, d5_2_shift)]
    for hw, w1, s1, b1, w2, s2, b2 in deep:
        d, xt = _deep_stage(xt, w1, s1, b1, w2, s2, b2, h=hw, w=hw, pool=True)
        downs.append(d.reshape(n, d.shape[1], hw, hw))

    c, _ = _deep_stage(xt, c1_w, c1_scale, c1_shift, c2_w, c2_scale, c2_shift,
                       h=8, w=8, pool=False)
    downs.append(c.reshape(n, 256, 8, 8))

    pooled6 = xt.reshape(n, 128, 8, 8).transpose(0, 2, 3, 1).astype(jnp.float32)
    out = _head(pooled6, (o1_w, o1_scale, o1_shift),
                (o2_w, o2_scale, o2_shift), (o3_w, o3_scale, o3_shift))
    return out, downs


# speedup vs baseline: 1.0018x; 1.0018x over previous
import functools

import jax
import jax.numpy as jnp
import numpy as np
from jax.experimental import pallas as pl
from jax.experimental.pallas import tpu as pltpu

_VMEM_LIMIT = 48 * 1024 * 1024


def _rup(v, m):
    return (v + m - 1) // m * m


def _band_stage_kernel(x_ref, bd1_ref, sh1_ref, bd2_ref, sh2_ref, e_ref,
                       q_ref, d_ref, p_ref, *, cin, cout, th, wid, hpa, nt):
    i = pl.program_id(1)
    xc = jnp.concatenate(
        [x_ref[0, pl.ds(ci * hpa + i * th, th + 4), :] for ci in range(cin)],
        axis=0)
    acc = None
    for dx in range(3):
        part = jnp.dot(bd1_ref[dx], xc[:, dx:dx + wid],
                       preferred_element_type=jnp.float32)
        acc = part if acc is None else acc + part
    y1 = jnp.maximum(acc + sh1_ref[0], 0.0).astype(jnp.bfloat16)
    zc = jnp.zeros((cout * (th + 2), 1), jnp.bfloat16)
    y1p = jnp.concatenate([zc, y1, zc], axis=1)
    acc2 = None
    for dx in range(3):
        part = jnp.dot(bd2_ref[dx], y1p[:, dx:dx + wid],
                       preferred_element_type=jnp.float32)
        acc2 = part if acc2 is None else acc2 + part
    y2 = jnp.maximum(acc2 + sh2_ref[...], 0.0)
    d_ref[0] = y2.reshape(cout, th, wid)
    y2b = y2.astype(jnp.bfloat16)
    m = jnp.maximum(
        jnp.dot(q_ref[0], y2b, preferred_element_type=jnp.float32),
        jnp.dot(q_ref[1], y2b, preferred_element_type=jnp.float32)
    ).astype(jnp.bfloat16)
    pooled = jnp.maximum(
        jnp.dot(m, e_ref[0], preferred_element_type=jnp.float32),
        jnp.dot(m, e_ref[1], preferred_element_type=jnp.float32))
    p_ref[0] = pooled.reshape(cout, th // 2, wid // 2).astype(jnp.bfloat16)


def _band_weights(w, scale, ro, ri, cin, cout, interleaved_cols):
    ws = w * scale[None, None, None, :]
    bds = []
    for dx in range(3):
        if interleaved_cols:
            acc = jnp.zeros((cout, ro, ri, cin), jnp.float32)
        else:
            acc = jnp.zeros((cout, ro, cin, ri), jnp.float32)
        for dy in range(3):
            e = jnp.eye(ro, ri, k=dy, dtype=jnp.float32)
            if interleaved_cols:
                acc = acc + (e[None, :, :, None]
                             * ws[dy, dx].T[:, None, None, :])
            else:
                acc = acc + (e[None, :, None, :]
                             * ws[dy, dx].T[:, None, :, None])
        bds.append(acc.reshape(cout * ro, cin * ri))
    return jnp.stack(bds).astype(jnp.bfloat16)


def _down_stage_band(x, w1, s1, b1, w2, s2, b2, *, cin, cout, h, th):
    n = x.shape[0]
    wid = h
    hpa = _rup(h + 4, 16)
    xp = jnp.pad(x.astype(jnp.bfloat16),
                 ((0, 0), (0, 0), (2, hpa - h - 2), (1, 1)))
    xp = xp.reshape(n, cin * hpa, wid + 2)
    bd1 = _band_weights(w1, s1, th + 2, th + 4, cin, cout, False)
    bd2 = _band_weights(w2, s2, th, th + 2, cout, cout, False)
    nt = h // th
    ro = (th + 2) * cout
    base = jnp.repeat(b1, th + 2).reshape(ro, 1)
    rl = np.arange(ro) % (th + 2)
    sh_first = jnp.where(jnp.asarray(rl == 0)[:, None], -1e30, base)
    sh_last = jnp.where(jnp.asarray(rl == th + 1)[:, None], -1e30, base)
    sh1 = jnp.concatenate([sh_first[None],
                           jnp.broadcast_to(base, (nt - 2, ro, 1)),
                           sh_last[None]], axis=0)
    sh2 = jnp.repeat(b2, th).reshape(-1, 1)
    we = np.zeros((2, wid, wid // 2), np.float32)
    we[0, np.arange(0, wid, 2), np.arange(wid // 2)] = 1
    we[1, np.arange(1, wid, 2), np.arange(wid // 2)] = 1
    e_sel = jnp.asarray(we, jnp.bfloat16)
    rq = np.zeros((2, cout * th // 2, cout * th), np.float32)
    rows = np.arange(cout * th // 2)
    src = (rows // (th // 2)) * th + 2 * (rows % (th // 2))
    rq[0, rows, src] = 1
    rq[1, rows, src + 1] = 1
    q_sel = jnp.asarray(rq, jnp.bfloat16)
    kfn = functools.partial(_band_stage_kernel, cin=cin, cout=cout, th=th,
                            wid=wid, hpa=hpa, nt=nt)
    d, p = pl.pallas_call(
        kfn,
        out_shape=(jax.ShapeDtypeStruct((n, cout, h, wid), jnp.float32),
                   jax.ShapeDtypeStruct((n, cout, h // 2, wid // 2),
                                        jnp.bfloat16)),
        grid=(n, nt),
        in_specs=[
            pl.BlockSpec((1, cin * hpa, wid + 2), lambda b, i: (b, 0, 0)),
            pl.BlockSpec((3, (th + 2) * cout, (th + 4) * cin),
                         lambda b, i: (0, 0, 0)),
            pl.BlockSpec((1, (th + 2) * cout, 1), lambda b, i: (i, 0, 0)),
            pl.BlockSpec((3, th * cout, (th + 2) * cout),
                         lambda b, i: (0, 0, 0)),
            pl.BlockSpec((th * cout, 1), lambda b, i: (0, 0)),
            pl.BlockSpec((2, wid, wid // 2), lambda b, i: (0, 0, 0)),
            pl.BlockSpec((2, cout * th // 2, cout * th),
                         lambda b, i: (0, 0, 0)),
        ],
        out_specs=(
            pl.BlockSpec((1, cout, th, wid), lambda b, i: (b, 0, i, 0)),
            pl.BlockSpec((1, cout, th // 2, wid // 2),
                         lambda b, i: (b, 0, i, 0)),
        ),
        compiler_params=pltpu.CompilerParams(
            dimension_semantics=("parallel", "arbitrary"),
            vmem_limit_bytes=_VMEM_LIMIT),
    )(xp, bd1, sh1, bd2, sh2, e_sel, q_sel)
    return d, p


def _flat_conv_t(xt, w_ref, sh_ref, col, *, h, w, cin, cout):
    m = h * w
    zp = jnp.zeros((cin, w + 1), jnp.bfloat16)
    xpf = jnp.concatenate([zp, xt, zp], axis=1)
    acc = None
    for dy in range(3):
        for dx in range(3):
            o = dy * w + dx
            sl = jax.lax.slice(xpf, (0, o), (cin, o + m))
            if dx == 0:
                sl = jnp.where(col == 0, jnp.bfloat16(0), sl)
            elif dx == 2:
                sl = jnp.where(col == w - 1, jnp.bfloat16(0), sl)
            part = jnp.dot(w_ref[3 * dy + dx], sl,
                           preferred_element_type=jnp.float32)
            acc = part if acc is None else acc + part
    return jnp.maximum(acc + sh_ref[...], 0.0)


def _deep_stage_kernel(x_ref, w1_ref, sh1_ref, w2_ref, sh2_ref, *rest,
                       h, w, cin, cout, pool):
    xt = x_ref[0]
    m = h * w
    col = jax.lax.broadcasted_iota(jnp.int32, (1, m), 1) % w
    y1 = _flat_conv_t(xt, w1_ref, sh1_ref, col,
                      h=h, w=w, cin=cin, cout=cout).astype(jnp.bfloat16)
    y2 = _flat_conv_t(y1, w2_ref, sh2_ref, col,
                      h=h, w=w, cin=cout, cout=cout)
    if pool:
        sel_ref, d_ref, p_ref = rest
        d_ref[0] = y2
        zc = jnp.zeros((cout, 1), jnp.float32)
        right = jnp.concatenate([jax.lax.slice(y2, (0, 1), (cout, m)), zc],
                                axis=1)
        mw = jnp.maximum(y2, right)
        zw = jnp.zeros((cout, w), jnp.float32)
        below = jnp.concatenate([jax.lax.slice(mw, (0, w), (cout, m)), zw],
                                axis=1)
        mh = jnp.maximum(mw, below)
        p_ref[0] = jnp.dot(mh.astype(jnp.bfloat16), sel_ref[...],
                           preferred_element_type=jnp.float32
                           ).astype(jnp.bfloat16)
    else:
        d_ref = rest[0]
        d_ref[0] = y2


def _pack_w_t(w, scale):
    ws = w * scale[None, None, None, :]
    return ws.reshape(9, w.shape[2], w.shape[3]).transpose(0, 2, 1).astype(
        jnp.bfloat16)


def _deep_stage(xt, w1, s1, b1, w2, s2, b2, *, h, w, pool):
    n, cin, m = xt.shape
    cout = w1.shape[-1]
    p1, p2 = _pack_w_t(w1, s1), _pack_w_t(w2, s2)
    sh1 = b1.reshape(cout, 1).astype(jnp.float32)
    sh2 = b2.reshape(cout, 1).astype(jnp.float32)
    out_shape = [jax.ShapeDtypeStruct((n, cout, m), jnp.float32)]
    out_specs = [pl.BlockSpec((1, cout, m), lambda b: (b, 0, 0))]
    in_specs = [
        pl.BlockSpec((1, cin, m), lambda b: (b, 0, 0)),
        pl.BlockSpec((9, cout, cin), lambda b: (0, 0, 0)),
        pl.BlockSpec((cout, 1), lambda b: (0, 0)),
        pl.BlockSpec((9, cout, cout), lambda b: (0, 0, 0)),
        pl.BlockSpec((cout, 1), lambda b: (0, 0)),
    ]
    args = [xt, p1, sh1, p2, sh2]
    if pool:
        out_shape.append(jax.ShapeDtypeStruct((n, cout, m // 4),
                                              jnp.bfloat16))
        out_specs.append(pl.BlockSpec((1, cout, m // 4), lambda b: (b, 0, 0)))
        in_specs.append(pl.BlockSpec((m, m // 4), lambda b: (0, 0)))
        q = np.arange(m // 4)
        p_src = 2 * (q // (w // 2)) * w + 2 * (q % (w // 2))
        selm = np.zeros((m, m // 4), np.float32)
        selm[p_src, q] = 1.0
        args.append(jnp.asarray(selm, jnp.bfloat16))
    kfn = functools.partial(_deep_stage_kernel, h=h, w=w, cin=cin, cout=cout,
                            pool=pool)
    outs = pl.pallas_call(
        kfn,
        out_shape=tuple(out_shape),
        grid=(n,),
        in_specs=in_specs,
        out_specs=tuple(out_specs),
        compiler_params=pltpu.CompilerParams(
            dimension_semantics=("parallel",),
            vmem_limit_bytes=_VMEM_LIMIT),
    )(*args)
    return (outs[0], outs[1]) if pool else (outs[0], None)


def _leaky(v):
    return jnp.where(v > 0, v, 0.2 * v)


def _head(x, o1, o2, o3):
    w, s, b = o1
    y = jax.lax.conv_general_dilated(
        x, w, (2, 2), ((1, 1), (1, 1)),
        dimension_numbers=("NHWC", "HWIO", "NHWC"))
    y = _leaky(y * s + b)
    w, s, b = o2
    y = jax.lax.conv_general_dilated(
        y, w, (1, 1), "VALID",
        dimension_numbers=("NHWC", "HWIO", "NHWC"))
    y = _leaky(y * s + b)
    w, s, b = o3
    y = jax.lax.conv_general_dilated(
        y, w, (1, 1), "VALID",
        dimension_numbers=("NHWC", "HWIO", "NHWC"))
    y = jax.nn.sigmoid(y * s + b)
    return jnp.transpose(y, (0, 3, 1, 2))


def kernel(img,
           d0_1_w, d0_1_scale, d0_1_shift, d0_2_w, d0_2_scale, d0_2_shift,
           d1_1_w, d1_1_scale, d1_1_shift, d1_2_w, d1_2_scale, d1_2_shift,
           d2_1_w, d2_1_scale, d2_1_shift, d2_2_w, d2_2_scale, d2_2_shift,
           d3_1_w, d3_1_scale, d3_1_shift, d3_2_w, d3_2_scale, d3_2_shift,
           d4_1_w, d4_1_scale, d4_1_shift, d4_2_w, d4_2_scale, d4_2_shift,
           d5_1_w, d5_1_scale, d5_1_shift, d5_2_w, d5_2_scale, d5_2_shift,
           c1_w, c1_scale, c1_shift, c2_w, c2_scale, c2_shift,
           o1_w, o1_scale, o1_shift, o2_w, o2_scale, o2_shift,
           o3_w, o3_scale, o3_shift):
    n = img.shape[0]
    downs = []

    band = [(3, 4, 512, 32, d0_1_w, d0_1_scale, d0_1_shift,
             d0_2_w, d0_2_scale, d0_2_shift),
            (4, 8, 256, 32, d1_1_w, d1_1_scale, d1_1_shift,
             d1_2_w, d1_2_scale, d1_2_shift),
            (8, 16, 128, 16, d2_1_w, d2_1_scale, d2_1_shift,
             d2_2_w, d2_2_scale, d2_2_shift)]
    x = img
    for cin, cout, h, th, w1, s1, b1, w2, s2, b2 in band:
        d, x = _down_stage_band(x, w1, s1, b1, w2, s2, b2,
                                cin=cin, cout=cout, h=h, th=th)
        downs.append(d)

    xt = x.reshape(n, 16, 64 * 64)
    deep = [(64, d3_1_w, d3_1_scale, d3_1_shift, d3_2_w, d3_2_scale, d3_2_shift),
            (32, d4_1_w, d4_1_scale, d4_1_shift, d4_2_w, d4_2_scale, d4_2_shift),
            (16, d5_1_w, d5_1_scale, d5_1_shift, d5_2_w, d5_2_scale, d5_2_shift)]
    for hw, w1, s1, b1, w2, s2, b2 in deep:
        d, xt = _deep_stage(xt, w1, s1, b1, w2, s2, b2, h=hw, w=hw, pool=True)
        downs.append(d.reshape(n, d.shape[1], hw, hw))

    c, _ = _deep_stage(xt, c1_w, c1_scale, c1_shift, c2_w, c2_scale, c2_shift,
                       h=8, w=8, pool=False)
    downs.append(c.reshape(n, 256, 8, 8))

    pooled6 = xt.reshape(n, 128, 8, 8).transpose(0, 2, 3, 1).astype(jnp.float32)
    out = _head(pooled6, (o1_w, o1_scale, o1_shift),
                (o2_w, o2_scale, o2_shift), (o3_w, o3_scale, o3_shift))
    return out, downs


# speedup vs baseline: 1.0658x; 1.0639x over previous
import functools

import jax
import jax.numpy as jnp
import numpy as np
from jax.experimental import pallas as pl
from jax.experimental.pallas import tpu as pltpu

_VMEM_LIMIT = 48 * 1024 * 1024


def _rup(v, m):
    return (v + m - 1) // m * m


def _band_stage_kernel(x_ref, bd1_ref, sh1_ref, bd2_ref, sh2_ref, e_ref,
                       q_ref, d_ref, p_ref, *, cin, cout, th, wid, hpa, nt):
    i = pl.program_id(1)
    xc = jnp.concatenate(
        [x_ref[0, pl.ds(ci * hpa + i * th, th + 4), :] for ci in range(cin)],
        axis=0)
    xk = jnp.concatenate([xc[:, dx:dx + wid] for dx in range(3)], axis=0)
    acc = jnp.dot(bd1_ref[0], xk, preferred_element_type=jnp.float32)
    y1 = jnp.maximum(acc + sh1_ref[0], 0.0).astype(jnp.bfloat16)
    zc = jnp.zeros((cout * (th + 2), 1), jnp.bfloat16)
    y1p = jnp.concatenate([zc, y1, zc], axis=1)
    yk = jnp.concatenate([y1p[:, dx:dx + wid] for dx in range(3)], axis=0)
    acc2 = jnp.dot(bd2_ref[0], yk, preferred_element_type=jnp.float32)
    y2 = jnp.maximum(acc2 + sh2_ref[...], 0.0)
    d_ref[0] = y2.reshape(cout, th, wid)
    y2b = y2.astype(jnp.bfloat16)
    m = jnp.maximum(
        jnp.dot(q_ref[0], y2b, preferred_element_type=jnp.float32),
        jnp.dot(q_ref[1], y2b, preferred_element_type=jnp.float32)
    ).astype(jnp.bfloat16)
    pooled = jnp.maximum(
        jnp.dot(m, e_ref[0], preferred_element_type=jnp.float32),
        jnp.dot(m, e_ref[1], preferred_element_type=jnp.float32))
    p_ref[0] = pooled.reshape(cout, th // 2, wid // 2).astype(jnp.bfloat16)


def _band_weights(w, scale, ro, ri, cin, cout, interleaved_cols):
    ws = w * scale[None, None, None, :]
    bds = []
    for dx in range(3):
        if interleaved_cols:
            acc = jnp.zeros((cout, ro, ri, cin), jnp.float32)
        else:
            acc = jnp.zeros((cout, ro, cin, ri), jnp.float32)
        for dy in range(3):
            e = jnp.eye(ro, ri, k=dy, dtype=jnp.float32)
            if interleaved_cols:
                acc = acc + (e[None, :, :, None]
                             * ws[dy, dx].T[:, None, None, :])
            else:
                acc = acc + (e[None, :, None, :]
                             * ws[dy, dx].T[:, None, :, None])
        bds.append(acc.reshape(cout * ro, cin * ri))
    return jnp.stack(bds).astype(jnp.bfloat16)


def _down_stage_band(x, w1, s1, b1, w2, s2, b2, *, cin, cout, h, th):
    n = x.shape[0]
    wid = h
    hpa = _rup(h + 4, 16)
    xp = jnp.pad(x.astype(jnp.bfloat16),
                 ((0, 0), (0, 0), (2, hpa - h - 2), (1, 1)))
    xp = xp.reshape(n, cin * hpa, wid + 2)
    bd1 = _band_weights(w1, s1, th + 2, th + 4, cin, cout, False)
    bd2 = _band_weights(w2, s2, th, th + 2, cout, cout, False)
    bd1 = jnp.concatenate([bd1[0], bd1[1], bd1[2]], axis=1)[None]
    bd2 = jnp.concatenate([bd2[0], bd2[1], bd2[2]], axis=1)[None]
    nt = h // th
    ro = (th + 2) * cout
    base = jnp.repeat(b1, th + 2).reshape(ro, 1)
    rl = np.arange(ro) % (th + 2)
    sh_first = jnp.where(jnp.asarray(rl == 0)[:, None], -1e30, base)
    sh_last = jnp.where(jnp.asarray(rl == th + 1)[:, None], -1e30, base)
    sh1 = jnp.concatenate([sh_first[None],
                           jnp.broadcast_to(base, (nt - 2, ro, 1)),
                           sh_last[None]], axis=0)
    sh2 = jnp.repeat(b2, th).reshape(-1, 1)
    we = np.zeros((2, wid, wid // 2), np.float32)
    we[0, np.arange(0, wid, 2), np.arange(wid // 2)] = 1
    we[1, np.arange(1, wid, 2), np.arange(wid // 2)] = 1
    e_sel = jnp.asarray(we, jnp.bfloat16)
    rq = np.zeros((2, cout * th // 2, cout * th), np.float32)
    rows = np.arange(cout * th // 2)
    src = (rows // (th // 2)) * th + 2 * (rows % (th // 2))
    rq[0, rows, src] = 1
    rq[1, rows, src + 1] = 1
    q_sel = jnp.asarray(rq, jnp.bfloat16)
    kfn = functools.partial(_band_stage_kernel, cin=cin, cout=cout, th=th,
                            wid=wid, hpa=hpa, nt=nt)
    d, p = pl.pallas_call(
        kfn,
        out_shape=(jax.ShapeDtypeStruct((n, cout, h, wid), jnp.float32),
                   jax.ShapeDtypeStruct((n, cout, h // 2, wid // 2),
                                        jnp.bfloat16)),
        grid=(n, nt),
        in_specs=[
            pl.BlockSpec((1, cin * hpa, wid + 2), lambda b, i: (b, 0, 0)),
            pl.BlockSpec((1, (th + 2) * cout, 3 * (th + 4) * cin),
                         lambda b, i: (0, 0, 0)),
            pl.BlockSpec((1, (th + 2) * cout, 1), lambda b, i: (i, 0, 0)),
            pl.BlockSpec((1, th * cout, 3 * (th + 2) * cout),
                         lambda b, i: (0, 0, 0)),
            pl.BlockSpec((th * cout, 1), lambda b, i: (0, 0)),
            pl.BlockSpec((2, wid, wid // 2), lambda b, i: (0, 0, 0)),
            pl.BlockSpec((2, cout * th // 2, cout * th),
                         lambda b, i: (0, 0, 0)),
        ],
        out_specs=(
            pl.BlockSpec((1, cout, th, wid), lambda b, i: (b, 0, i, 0)),
            pl.BlockSpec((1, cout, th // 2, wid // 2),
                         lambda b, i: (b, 0, i, 0)),
        ),
        compiler_params=pltpu.CompilerParams(
            dimension_semantics=("parallel", "arbitrary"),
            vmem_limit_bytes=_VMEM_LIMIT),
    )(xp, bd1, sh1, bd2, sh2, e_sel, q_sel)
    return d, p


def _flat_conv_t(xt, w_ref, sh_ref, col, *, h, w, cin, cout):
    m = h * w
    zp = jnp.zeros((cin, w + 1), jnp.bfloat16)
    xpf = jnp.concatenate([zp, xt, zp], axis=1)
    acc = None
    for dy in range(3):
        for dx in range(3):
            o = dy * w + dx
            sl = jax.lax.slice(xpf, (0, o), (cin, o + m))
            if dx == 0:
                sl = jnp.where(col == 0, jnp.bfloat16(0), sl)
            elif dx == 2:
                sl = jnp.where(col == w - 1, jnp.bfloat16(0), sl)
            part = jnp.dot(w_ref[3 * dy + dx], sl,
                           preferred_element_type=jnp.float32)
            acc = part if acc is None else acc + part
    return jnp.maximum(acc + sh_ref[...], 0.0)


def _deep_stage_kernel(x_ref, w1_ref, sh1_ref, w2_ref, sh2_ref, *rest,
                       h, w, cin, cout, pool):
    xt = x_ref[0]
    m = h * w
    col = jax.lax.broadcasted_iota(jnp.int32, (1, m), 1) % w
    y1 = _flat_conv_t(xt, w1_ref, sh1_ref, col,
                      h=h, w=w, cin=cin, cout=cout).astype(jnp.bfloat16)
    y2 = _flat_conv_t(y1, w2_ref, sh2_ref, col,
                      h=h, w=w, cin=cout, cout=cout)
    if pool:
        sel_ref, d_ref, p_ref = rest
        d_ref[0] = y2
        zc = jnp.zeros((cout, 1), jnp.float32)
        right = jnp.concatenate([jax.lax.slice(y2, (0, 1), (cout, m)), zc],
                                axis=1)
        mw = jnp.maximum(y2, right)
        zw = jnp.zeros((cout, w), jnp.float32)
        below = jnp.concatenate([jax.lax.slice(mw, (0, w), (cout, m)), zw],
                                axis=1)
        mh = jnp.maximum(mw, below)
        p_ref[0] = jnp.dot(mh.astype(jnp.bfloat16), sel_ref[...],
                           preferred_element_type=jnp.float32
                           ).astype(jnp.bfloat16)
    else:
        d_ref = rest[0]
        d_ref[0] = y2


def _pack_w_t(w, scale):
    ws = w * scale[None, None, None, :]
    return ws.reshape(9, w.shape[2], w.shape[3]).transpose(0, 2, 1).astype(
        jnp.bfloat16)


def _deep_stage(xt, w1, s1, b1, w2, s2, b2, *, h, w, pool):
    n, cin, m = xt.shape
    cout = w1.shape[-1]
    p1, p2 = _pack_w_t(w1, s1), _pack_w_t(w2, s2)
    sh1 = b1.reshape(cout, 1).astype(jnp.float32)
    sh2 = b2.reshape(cout, 1).astype(jnp.float32)
    out_shape = [jax.ShapeDtypeStruct((n, cout, m), jnp.float32)]
    out_specs = [pl.BlockSpec((1, cout, m), lambda b: (b, 0, 0))]
    in_specs = [
        pl.BlockSpec((1, cin, m), lambda b: (b, 0, 0)),
        pl.BlockSpec((9, cout, cin), lambda b: (0, 0, 0)),
        pl.BlockSpec((cout, 1), lambda b: (0, 0)),
        pl.BlockSpec((9, cout, cout), lambda b: (0, 0, 0)),
        pl.BlockSpec((cout, 1), lambda b: (0, 0)),
    ]
    args = [xt, p1, sh1, p2, sh2]
    if pool:
        out_shape.append(jax.ShapeDtypeStruct((n, cout, m // 4),
                                              jnp.bfloat16))
        out_specs.append(pl.BlockSpec((1, cout, m // 4), lambda b: (b, 0, 0)))
        in_specs.append(pl.BlockSpec((m, m // 4), lambda b: (0, 0)))
        q = np.arange(m // 4)
        p_src = 2 * (q // (w // 2)) * w + 2 * (q % (w // 2))
        selm = np.zeros((m, m // 4), np.float32)
        selm[p_src, q] = 1.0
        args.append(jnp.asarray(selm, jnp.bfloat16))
    kfn = functools.partial(_deep_stage_kernel, h=h, w=w, cin=cin, cout=cout,
                            pool=pool)
    outs = pl.pallas_call(
        kfn,
        out_shape=tuple(out_shape),
        grid=(n,),
        in_specs=in_specs,
        out_specs=tuple(out_specs),
        compiler_params=pltpu.CompilerParams(
            dimension_semantics=("parallel",),
            vmem_limit_bytes=_VMEM_LIMIT),
    )(*args)
    return (outs[0], outs[1]) if pool else (outs[0], None)


def _leaky(v):
    return jnp.where(v > 0, v, 0.2 * v)


def _head(x, o1, o2, o3):
    w, s, b = o1
    y = jax.lax.conv_general_dilated(
        x, w, (2, 2), ((1, 1), (1, 1)),
        dimension_numbers=("NHWC", "HWIO", "NHWC"))
    y = _leaky(y * s + b)
    w, s, b = o2
    y = jax.lax.conv_general_dilated(
        y, w, (1, 1), "VALID",
        dimension_numbers=("NHWC", "HWIO", "NHWC"))
    y = _leaky(y * s + b)
    w, s, b = o3
    y = jax.lax.conv_general_dilated(
        y, w, (1, 1), "VALID",
        dimension_numbers=("NHWC", "HWIO", "NHWC"))
    y = jax.nn.sigmoid(y * s + b)
    return jnp.transpose(y, (0, 3, 1, 2))


def kernel(img,
           d0_1_w, d0_1_scale, d0_1_shift, d0_2_w, d0_2_scale, d0_2_shift,
           d1_1_w, d1_1_scale, d1_1_shift, d1_2_w, d1_2_scale, d1_2_shift,
           d2_1_w, d2_1_scale, d2_1_shift, d2_2_w, d2_2_scale, d2_2_shift,
           d3_1_w, d3_1_scale, d3_1_shift, d3_2_w, d3_2_scale, d3_2_shift,
           d4_1_w, d4_1_scale, d4_1_shift, d4_2_w, d4_2_scale, d4_2_shift,
           d5_1_w, d5_1_scale, d5_1_shift, d5_2_w, d5_2_scale, d5_2_shift,
           c1_w, c1_scale, c1_shift, c2_w, c2_scale, c2_shift,
           o1_w, o1_scale, o1_shift, o2_w, o2_scale, o2_shift,
           o3_w, o3_scale, o3_shift):
    n = img.shape[0]
    downs = []

    band = [(3, 4, 512, 32, d0_1_w, d0_1_scale, d0_1_shift,
             d0_2_w, d0_2_scale, d0_2_shift),
            (4, 8, 256, 32, d1_1_w, d1_1_scale, d1_1_shift,
             d1_2_w, d1_2_scale, d1_2_shift),
            (8, 16, 128, 16, d2_1_w, d2_1_scale, d2_1_shift,
             d2_2_w, d2_2_scale, d2_2_shift)]
    x = img
    for cin, cout, h, th, w1, s1, b1, w2, s2, b2 in band:
        d, x = _down_stage_band(x, w1, s1, b1, w2, s2, b2,
                                cin=cin, cout=cout, h=h, th=th)
        downs.append(d)

    xt = x.reshape(n, 16, 64 * 64)
    deep = [(64, d3_1_w, d3_1_scale, d3_1_shift, d3_2_w, d3_2_scale, d3_2_shift),
            (32, d4_1_w, d4_1_scale, d4_1_shift, d4_2_w, d4_2_scale, d4_2_shift),
            (16, d5_1_w, d5_1_scale, d5_1_shift, d5_2_w, d5_2_scale, d5_2_shift)]
    for hw, w1, s1, b1, w2, s2, b2 in deep:
        d, xt = _deep_stage(xt, w1, s1, b1, w2, s2, b2, h=hw, w=hw, pool=True)
        downs.append(d.reshape(n, d.shape[1], hw, hw))

    c, _ = _deep_stage(xt, c1_w, c1_scale, c1_shift, c2_w, c2_scale, c2_shift,
                       h=8, w=8, pool=False)
    downs.append(c.reshape(n, 256, 8, 8))

    pooled6 = xt.reshape(n, 128, 8, 8).transpose(0, 2, 3, 1).astype(jnp.float32)
    out = _head(pooled6, (o1_w, o1_scale, o1_shift),
                (o2_w, o2_scale, o2_shift), (o3_w, o3_scale, o3_shift))
    return out, downs


# speedup vs baseline: 1.0891x; 1.0219x over previous
import functools

import jax
import jax.numpy as jnp
import numpy as np
from jax.experimental import pallas as pl
from jax.experimental.pallas import tpu as pltpu

_VMEM_LIMIT = 48 * 1024 * 1024


def _rup(v, m):
    return (v + m - 1) // m * m


def _band_stage_kernel(x_ref, bd1_ref, sh1_ref, bd2_ref, sh2_ref, e_ref,
                       q_ref, d_ref, p_ref, *, cin, cout, th, wid, hpa, nt):
    i = pl.program_id(1)
    xc = jnp.concatenate(
        [x_ref[0, pl.ds(ci * hpa + i * th, th + 4), :] for ci in range(cin)],
        axis=0)
    xk = jnp.concatenate([xc[:, dx:dx + wid] for dx in range(3)], axis=0)
    acc = jnp.dot(bd1_ref[0], xk, preferred_element_type=jnp.float32)
    y1 = jnp.maximum(acc + sh1_ref[0], 0.0).astype(jnp.bfloat16)
    zc = jnp.zeros((cout * (th + 2), 1), jnp.bfloat16)
    y1p = jnp.concatenate([zc, y1, zc], axis=1)
    yk = jnp.concatenate([y1p[:, dx:dx + wid] for dx in range(3)], axis=0)
    acc2 = jnp.dot(bd2_ref[0], yk, preferred_element_type=jnp.float32)
    y2 = jnp.maximum(acc2 + sh2_ref[...], 0.0)
    d_ref[0] = y2.reshape(cout, th, wid)
    y2b = y2.astype(jnp.bfloat16)
    m = jnp.maximum(
        jnp.dot(q_ref[0], y2b, preferred_element_type=jnp.float32),
        jnp.dot(q_ref[1], y2b, preferred_element_type=jnp.float32)
    ).astype(jnp.bfloat16)
    pooled = jnp.maximum(
        jnp.dot(m, e_ref[0], preferred_element_type=jnp.float32),
        jnp.dot(m, e_ref[1], preferred_element_type=jnp.float32))
    p_ref[0] = pooled.reshape(cout, th // 2, wid // 2).astype(jnp.bfloat16)


def _band_weights(w, scale, ro, ri, cin, cout, interleaved_cols):
    ws = w * scale[None, None, None, :]
    bds = []
    for dx in range(3):
        if interleaved_cols:
            acc = jnp.zeros((cout, ro, ri, cin), jnp.float32)
        else:
            acc = jnp.zeros((cout, ro, cin, ri), jnp.float32)
        for dy in range(3):
            e = jnp.eye(ro, ri, k=dy, dtype=jnp.float32)
            if interleaved_cols:
                acc = acc + (e[None, :, :, None]
                             * ws[dy, dx].T[:, None, None, :])
            else:
                acc = acc + (e[None, :, None, :]
                             * ws[dy, dx].T[:, None, :, None])
        bds.append(acc.reshape(cout * ro, cin * ri))
    return jnp.stack(bds).astype(jnp.bfloat16)


def _down_stage_band(x, w1, s1, b1, w2, s2, b2, *, cin, cout, h, th):
    n = x.shape[0]
    wid = h
    hpa = _rup(h + 4, 16)
    xp = jnp.pad(x.astype(jnp.bfloat16),
                 ((0, 0), (0, 0), (2, hpa - h - 2), (1, 1)))
    xp = xp.reshape(n, cin * hpa, wid + 2)
    bd1 = _band_weights(w1, s1, th + 2, th + 4, cin, cout, False)
    bd2 = _band_weights(w2, s2, th, th + 2, cout, cout, False)
    bd1 = jnp.concatenate([bd1[0], bd1[1], bd1[2]], axis=1)[None]
    bd2 = jnp.concatenate([bd2[0], bd2[1], bd2[2]], axis=1)[None]
    nt = h // th
    ro = (th + 2) * cout
    base = jnp.repeat(b1, th + 2).reshape(ro, 1)
    rl = np.arange(ro) % (th + 2)
    sh_first = jnp.where(jnp.asarray(rl == 0)[:, None], -1e30, base)
    sh_last = jnp.where(jnp.asarray(rl == th + 1)[:, None], -1e30, base)
    sh1 = jnp.concatenate([sh_first[None],
                           jnp.broadcast_to(base, (nt - 2, ro, 1)),
                           sh_last[None]], axis=0)
    sh2 = jnp.repeat(b2, th).reshape(-1, 1)
    we = np.zeros((2, wid, wid // 2), np.float32)
    we[0, np.arange(0, wid, 2), np.arange(wid // 2)] = 1
    we[1, np.arange(1, wid, 2), np.arange(wid // 2)] = 1
    e_sel = jnp.asarray(we, jnp.bfloat16)
    rq = np.zeros((2, cout * th // 2, cout * th), np.float32)
    rows = np.arange(cout * th // 2)
    src = (rows // (th // 2)) * th + 2 * (rows % (th // 2))
    rq[0, rows, src] = 1
    rq[1, rows, src + 1] = 1
    q_sel = jnp.asarray(rq, jnp.bfloat16)
    kfn = functools.partial(_band_stage_kernel, cin=cin, cout=cout, th=th,
                            wid=wid, hpa=hpa, nt=nt)
    d, p = pl.pallas_call(
        kfn,
        out_shape=(jax.ShapeDtypeStruct((n, cout, h, wid), jnp.float32),
                   jax.ShapeDtypeStruct((n, cout, h // 2, wid // 2),
                                        jnp.bfloat16)),
        grid=(n, nt),
        in_specs=[
            pl.BlockSpec((1, cin * hpa, wid + 2), lambda b, i: (b, 0, 0)),
            pl.BlockSpec((1, (th + 2) * cout, 3 * (th + 4) * cin),
                         lambda b, i: (0, 0, 0)),
            pl.BlockSpec((1, (th + 2) * cout, 1), lambda b, i: (i, 0, 0)),
            pl.BlockSpec((1, th * cout, 3 * (th + 2) * cout),
                         lambda b, i: (0, 0, 0)),
            pl.BlockSpec((th * cout, 1), lambda b, i: (0, 0)),
            pl.BlockSpec((2, wid, wid // 2), lambda b, i: (0, 0, 0)),
            pl.BlockSpec((2, cout * th // 2, cout * th),
                         lambda b, i: (0, 0, 0)),
        ],
        out_specs=(
            pl.BlockSpec((1, cout, th, wid), lambda b, i: (b, 0, i, 0)),
            pl.BlockSpec((1, cout, th // 2, wid // 2),
                         lambda b, i: (b, 0, i, 0)),
        ),
        compiler_params=pltpu.CompilerParams(
            dimension_semantics=("parallel", "arbitrary"),
            vmem_limit_bytes=_VMEM_LIMIT),
    )(xp, bd1, sh1, bd2, sh2, e_sel, q_sel)
    return d, p


def _flat_conv_t(xt, w_ref, sh_ref, col, *, h, w, cin, cout):
    m = h * w
    zp = jnp.zeros((cin, w + 1), jnp.bfloat16)
    xpf = jnp.concatenate([zp, xt, zp], axis=1)
    sls = []
    for dy in range(3):
        for dx in range(3):
            o = dy * w + dx
            sl = jax.lax.slice(xpf, (0, o), (cin, o + m))
            if dx == 0:
                sl = jnp.where(col == 0, jnp.bfloat16(0), sl)
            elif dx == 2:
                sl = jnp.where(col == w - 1, jnp.bfloat16(0), sl)
            sls.append(sl)
    xk = jnp.concatenate(sls, axis=0)
    acc = jnp.dot(w_ref[0], xk, preferred_element_type=jnp.float32)
    return jnp.maximum(acc + sh_ref[...], 0.0)


def _deep_stage_kernel(x_ref, w1_ref, sh1_ref, w2_ref, sh2_ref, *rest,
                       h, w, cin, cout, pool):
    xt = x_ref[0]
    m = h * w
    col = jax.lax.broadcasted_iota(jnp.int32, (1, m), 1) % w
    y1 = _flat_conv_t(xt, w1_ref, sh1_ref, col,
                      h=h, w=w, cin=cin, cout=cout).astype(jnp.bfloat16)
    y2 = _flat_conv_t(y1, w2_ref, sh2_ref, col,
                      h=h, w=w, cin=cout, cout=cout)
    if pool:
        sel_ref, d_ref, p_ref = rest
        d_ref[0] = y2
        zc = jnp.zeros((cout, 1), jnp.float32)
        right = jnp.concatenate([jax.lax.slice(y2, (0, 1), (cout, m)), zc],
                                axis=1)
        mw = jnp.maximum(y2, right)
        zw = jnp.zeros((cout, w), jnp.float32)
        below = jnp.concatenate([jax.lax.slice(mw, (0, w), (cout, m)), zw],
                                axis=1)
        mh = jnp.maximum(mw, below)
        p_ref[0] = jnp.dot(mh.astype(jnp.bfloat16), sel_ref[...],
                           preferred_element_type=jnp.float32
                           ).astype(jnp.bfloat16)
    else:
        d_ref = rest[0]
        d_ref[0] = y2


def _pack_w_t(w, scale):
    ws = w * scale[None, None, None, :]
    cin, cout = w.shape[2], w.shape[3]
    return ws.reshape(9, cin, cout).transpose(2, 0, 1).reshape(
        cout, 9 * cin)[None].astype(jnp.bfloat16)


def _deep_stage(xt, w1, s1, b1, w2, s2, b2, *, h, w, pool):
    n, cin, m = xt.shape
    cout = w1.shape[-1]
    p1, p2 = _pack_w_t(w1, s1), _pack_w_t(w2, s2)
    sh1 = b1.reshape(cout, 1).astype(jnp.float32)
    sh2 = b2.reshape(cout, 1).astype(jnp.float32)
    out_shape = [jax.ShapeDtypeStruct((n, cout, m), jnp.float32)]
    out_specs = [pl.BlockSpec((1, cout, m), lambda b: (b, 0, 0))]
    in_specs = [
        pl.BlockSpec((1, cin, m), lambda b: (b, 0, 0)),
        pl.BlockSpec((1, cout, 9 * cin), lambda b: (0, 0, 0)),
        pl.BlockSpec((cout, 1), lambda b: (0, 0)),
        pl.BlockSpec((1, cout, 9 * cout), lambda b: (0, 0, 0)),
        pl.BlockSpec((cout, 1), lambda b: (0, 0)),
    ]
    args = [xt, p1, sh1, p2, sh2]
    if pool:
        out_shape.append(jax.ShapeDtypeStruct((n, cout, m // 4),
                                              jnp.bfloat16))
        out_specs.append(pl.BlockSpec((1, cout, m // 4), lambda b: (b, 0, 0)))
        in_specs.append(pl.BlockSpec((m, m // 4), lambda b: (0, 0)))
        q = np.arange(m // 4)
        p_src = 2 * (q // (w // 2)) * w + 2 * (q % (w // 2))
        selm = np.zeros((m, m // 4), np.float32)
        selm[p_src, q] = 1.0
        args.append(jnp.asarray(selm, jnp.bfloat16))
    kfn = functools.partial(_deep_stage_kernel, h=h, w=w, cin=cin, cout=cout,
                            pool=pool)
    outs = pl.pallas_call(
        kfn,
        out_shape=tuple(out_shape),
        grid=(n,),
        in_specs=in_specs,
        out_specs=tuple(out_specs),
        compiler_params=pltpu.CompilerParams(
            dimension_semantics=("parallel",),
            vmem_limit_bytes=_VMEM_LIMIT),
    )(*args)
    return (outs[0], outs[1]) if pool else (outs[0], None)


def _leaky(v):
    return jnp.where(v > 0, v, 0.2 * v)


def _head(x, o1, o2, o3):
    w, s, b = o1
    y = jax.lax.conv_general_dilated(
        x, w, (2, 2), ((1, 1), (1, 1)),
        dimension_numbers=("NHWC", "HWIO", "NHWC"))
    y = _leaky(y * s + b)
    w, s, b = o2
    y = jax.lax.conv_general_dilated(
        y, w, (1, 1), "VALID",
        dimension_numbers=("NHWC", "HWIO", "NHWC"))
    y = _leaky(y * s + b)
    w, s, b = o3
    y = jax.lax.conv_general_dilated(
        y, w, (1, 1), "VALID",
        dimension_numbers=("NHWC", "HWIO", "NHWC"))
    y = jax.nn.sigmoid(y * s + b)
    return jnp.transpose(y, (0, 3, 1, 2))


def kernel(img,
           d0_1_w, d0_1_scale, d0_1_shift, d0_2_w, d0_2_scale, d0_2_shift,
           d1_1_w, d1_1_scale, d1_1_shift, d1_2_w, d1_2_scale, d1_2_shift,
           d2_1_w, d2_1_scale, d2_1_shift, d2_2_w, d2_2_scale, d2_2_shift,
           d3_1_w, d3_1_scale, d3_1_shift, d3_2_w, d3_2_scale, d3_2_shift,
           d4_1_w, d4_1_scale, d4_1_shift, d4_2_w, d4_2_scale, d4_2_shift,
           d5_1_w, d5_1_scale, d5_1_shift, d5_2_w, d5_2_scale, d5_2_shift,
           c1_w, c1_scale, c1_shift, c2_w, c2_scale, c2_shift,
           o1_w, o1_scale, o1_shift, o2_w, o2_scale, o2_shift,
           o3_w, o3_scale, o3_shift):
    n = img.shape[0]
    downs = []

    band = [(3, 4, 512, 32, d0_1_w, d0_1_scale, d0_1_shift,
             d0_2_w, d0_2_scale, d0_2_shift),
            (4, 8, 256, 32, d1_1_w, d1_1_scale, d1_1_shift,
             d1_2_w, d1_2_scale, d1_2_shift),
            (8, 16, 128, 16, d2_1_w, d2_1_scale, d2_1_shift,
             d2_2_w, d2_2_scale, d2_2_shift)]
    x = img
    for cin, cout, h, th, w1, s1, b1, w2, s2, b2 in band:
        d, x = _down_stage_band(x, w1, s1, b1, w2, s2, b2,
                                cin=cin, cout=cout, h=h, th=th)
        downs.append(d)

    xt = x.reshape(n, 16, 64 * 64)
    deep = [(64, d3_1_w, d3_1_scale, d3_1_shift, d3_2_w, d3_2_scale, d3_2_shift),
            (32, d4_1_w, d4_1_scale, d4_1_shift, d4_2_w, d4_2_scale, d4_2_shift),
            (16, d5_1_w, d5_1_scale, d5_1_shift, d5_2_w, d5_2_scale, d5_2_shift)]
    for hw, w1, s1, b1, w2, s2, b2 in deep:
        d, xt = _deep_stage(xt, w1, s1, b1, w2, s2, b2, h=hw, w=hw, pool=True)
        downs.append(d.reshape(n, d.shape[1], hw, hw))

    c, _ = _deep_stage(xt, c1_w, c1_scale, c1_shift, c2_w, c2_scale, c2_shift,
                       h=8, w=8, pool=False)
    downs.append(c.reshape(n, 256, 8, 8))

    pooled6 = xt.reshape(n, 128, 8, 8).transpose(0, 2, 3, 1).astype(jnp.float32)
    out = _head(pooled6, (o1_w, o1_scale, o1_shift),
                (o2_w, o2_scale, o2_shift), (o3_w, o3_scale, o3_shift))
    return out, downs
